# baseline (device time: 31297 ns/iter reference)
import jax
import jax.numpy as jnp
from jax import lax
from jax.experimental import pallas as pl
from jax.experimental.pallas import tpu as pltpu

N_DEV = 8
M_PER = 512
K = 4096
N = 2048
N_PER = N // N_DEV
W_SLOTS = 4
PREFETCH = 2


def kernel(x, w_mat, scale_x, scale_w):
    def body(x_hbm, w_hbm, sx_ref, sw_ref, out_hbm,
             xbuf, wbuf, sendbuf, recvbuf, ystage,
             x_sem, w_sems, out_sems, send_sems, recv_sems):
        me = lax.axis_index("i")
        s = sx_ref[0] * sw_ref[0]

        x_cp = pltpu.make_async_copy(x_hbm, xbuf, x_sem)
        x_cp.start()

        def w_load(step):
            dst = (me + 1 + step) % N_DEV
            slot = step % W_SLOTS
            return pltpu.make_async_copy(
                w_hbm.at[:, pl.ds(dst * N_PER, N_PER)],
                wbuf.at[slot],
                w_sems.at[slot],
            )

        loads = {}
        for st in range(PREFETCH):
            loads[st] = w_load(st)
            loads[st].start()

        barrier_sem = pltpu.get_barrier_semaphore()
        for k in range(1, N_DEV):
            pl.semaphore_signal(
                barrier_sem, inc=1,
                device_id=((me + k) % N_DEV,),
                device_id_type=pl.DeviceIdType.MESH,
            )
        pl.semaphore_wait(barrier_sem, N_DEV - 1)

        x_cp.wait()
        xv = xbuf[...].astype(jnp.float8_e4m3fn)

        out_cps = []

        def store_rows(row_pos, yv32):
            ystage[row_pos] = yv32
            cp = pltpu.make_async_copy(
                ystage.at[row_pos],
                out_hbm.at[pl.ds(row_pos * M_PER, M_PER), :],
                out_sems.at[row_pos],
            )
            cp.start()
            out_cps.append(cp)

        def drain(k):
            src = (me - k) % N_DEV
            recv = pltpu.make_async_remote_copy(
                src_ref=sendbuf.at[0],
                dst_ref=recvbuf.at[src],
                send_sem=send_sems.at[0],
                recv_sem=recv_sems.at[src],
                device_id=(me,),
                device_id_type=pl.DeviceIdType.MESH,
            )
            recv.wait_recv()
            store_rows(src, recvbuf[src].astype(jnp.float32))

        rdmas = []
        for step in range(N_DEV):
            loads[step].wait()
            nxt = step + PREFETCH
            if nxt < N_DEV:
                loads[nxt] = w_load(nxt)
                loads[nxt].start()

            wj = wbuf[step % W_SLOTS].astype(jnp.float8_e4m3fn)
            acc = jnp.dot(xv, wj, preferred_element_type=jnp.float32)
            yj = jnp.maximum(acc * s, 0.0)

            if step < N_DEV - 1:
                dst = (me + 1 + step) % N_DEV
                sendbuf[step] = yj.astype(jnp.bfloat16)
                rdma = pltpu.make_async_remote_copy(
                    src_ref=sendbuf.at[step],
                    dst_ref=recvbuf.at[me],
                    send_sem=send_sems.at[step],
                    recv_sem=recv_sems.at[me],
                    device_id=(dst,),
                    device_id_type=pl.DeviceIdType.MESH,
                )
                rdma.start()
                rdmas.append(rdma)
            else:
                store_rows_me = yj
                store_rows(me, store_rows_me)

        for k in range(1, N_DEV):
            drain(k)

        for cp in out_cps:
            cp.wait()
        for rdma in rdmas:
            rdma.wait_send()

    out_shape = jax.ShapeDtypeStruct((N_DEV * M_PER, N_PER), jnp.float32)
    return pl.pallas_call(
        body,
        out_shape=out_shape,
        in_specs=[
            pl.BlockSpec(memory_space=pl.ANY),
            pl.BlockSpec(memory_space=pl.ANY),
            pl.BlockSpec(memory_space=pltpu.SMEM),
            pl.BlockSpec(memory_space=pltpu.SMEM),
        ],
        out_specs=pl.BlockSpec(memory_space=pl.ANY),
        scratch_shapes=[
            pltpu.VMEM((M_PER, K), jnp.float32),
            pltpu.VMEM((W_SLOTS, K, N_PER), jnp.float32),
            pltpu.VMEM((N_DEV - 1, M_PER, N_PER), jnp.bfloat16),
            pltpu.VMEM((N_DEV, M_PER, N_PER), jnp.bfloat16),
            pltpu.VMEM((N_DEV, M_PER, N_PER), jnp.float32),
            pltpu.SemaphoreType.DMA,
            pltpu.SemaphoreType.DMA((W_SLOTS,)),
            pltpu.SemaphoreType.DMA((N_DEV,)),
            pltpu.SemaphoreType.DMA((N_DEV - 1,)),
            pltpu.SemaphoreType.DMA((N_DEV,)),
        ],
        compiler_params=pltpu.CompilerParams(
            vmem_limit_bytes=110 * 1024 * 1024,
            collective_id=0,
        ),
    )(x, w_mat, scale_x, scale_w)
